# baseline (speedup 1.0000x reference)
import jax
import jax.numpy as jnp
from jax import lax
from jax.experimental import pallas as pl
from jax.experimental.pallas import tpu as pltpu
from jax.experimental.pallas import tpu_sc as plsc

N_NODES = 10000
N_PAD = 10240
N_EDGES = 320000
D = 128
DH = D // 2

NC = 2
NS = 16
E_PER_S = N_EDGES // NS
CHUNK = 80
N_CHUNKS = E_PER_S // CHUNK
ROWS_PER_TILE = N_PAD // NS
CNT_W = 16


def _sc_kernel(xlo_hbm, xhi_hbm, src_hbm, dst_hbm, acc_out, cnt_out,
               src_all, dst_all, rows0, rows1, ones_v, zacc, zcnt,
               acc_sh, cnt_sh, gsem0, gsem1, ssem0, ssem1, csem):
    cid = lax.axis_index("c")
    sid = lax.axis_index("s")

    zero16 = jnp.zeros((16,), jnp.float32)
    one16 = jnp.ones((16,), jnp.float32)

    def init_zacc(i, carry):
        for c8 in range(DH // 16):
            zacc[i, pl.ds(c8 * 16, 16)] = zero16
        return carry

    lax.fori_loop(0, 128, init_zacc, 0)

    def init_zcnt(i, carry):
        zcnt[i, :] = zero16
        return carry

    lax.fori_loop(0, ROWS_PER_TILE, init_zcnt, 0)

    def init_ones(i, carry):
        ones_v[i, :] = one16
        return carry

    lax.fori_loop(0, CHUNK, init_ones, 0)

    for b in range(ROWS_PER_TILE // 128):
        pltpu.sync_copy(zacc, acc_sh.at[pl.ds(sid * ROWS_PER_TILE + b * 128, 128)])
    pltpu.sync_copy(zcnt, cnt_sh.at[pl.ds(sid * ROWS_PER_TILE, ROWS_PER_TILE)])

    plsc.subcore_barrier()

    ebase = sid * E_PER_S

    pltpu.sync_copy(src_hbm.at[pl.ds(ebase, E_PER_S)], src_all)
    pltpu.sync_copy(dst_hbm.at[pl.ds(ebase, E_PER_S)], dst_all)

    def gather(i, buf, sem):
        idx = src_all.at[pl.ds(i * CHUNK, CHUNK)]

        @pl.when(cid == 0)
        def _():
            pltpu.async_copy(xlo_hbm.at[idx], buf, sem)

        @pl.when(cid == 1)
        def _():
            pltpu.async_copy(xhi_hbm.at[idx], buf, sem)

    def gwait(buf, sem):
        pltpu.make_async_copy(xlo_hbm.at[pl.ds(0, CHUNK)], buf, sem).wait()

    def acc_scatter(i, buf, sem):
        didx = dst_all.at[pl.ds(i * CHUNK, CHUNK)]
        pltpu.async_copy(buf, acc_sh.at[didx], sem, add=True)

    def acc_swait(i, buf, sem):
        didx = dst_all.at[pl.ds(i * CHUNK, CHUNK)]
        pltpu.make_async_copy(buf, acc_sh.at[didx], sem).wait()

    def cnt_scatter(i, sem):
        didx = dst_all.at[pl.ds(i * CHUNK, CHUNK)]
        pltpu.async_copy(ones_v, cnt_sh.at[didx], sem, add=True)

    def cnt_swait(i, sem):
        didx = dst_all.at[pl.ds(i * CHUNK, CHUNK)]
        pltpu.make_async_copy(ones_v, cnt_sh.at[didx], sem).wait()

    gather(0, rows0, gsem0)

    def half(i, buf, gsem, ssem, obuf, ogsem, ossem):
        gwait(buf, gsem)

        @pl.when(i > 0)
        def _():
            acc_swait(i - 1, obuf, ossem)

            @pl.when(cid == 0)
            def _():
                cnt_swait(i - 1, csem)

        @pl.when(i + 1 < N_CHUNKS)
        def _():
            gather(i + 1, obuf, ogsem)

        acc_scatter(i, buf, ssem)

        @pl.when(cid == 0)
        def _():
            cnt_scatter(i, csem)

    def two_chunks(j, carry):
        half(2 * j, rows0, gsem0, ssem0, rows1, gsem1, ssem1)
        half(2 * j + 1, rows1, gsem1, ssem1, rows0, gsem0, ssem0)
        return carry

    lax.fori_loop(0, N_CHUNKS // 2, two_chunks, 0)
    acc_swait(N_CHUNKS - 1, rows1, ssem1)

    @pl.when(cid == 0)
    def _():
        cnt_swait(N_CHUNKS - 1, csem)

    plsc.subcore_barrier()

    rbase = sid * ROWS_PER_TILE
    pltpu.sync_copy(acc_sh.at[pl.ds(rbase, ROWS_PER_TILE)],
                    acc_out.at[cid, pl.ds(rbase, ROWS_PER_TILE)])

    @pl.when(cid == 0)
    def _():
        pltpu.sync_copy(cnt_sh.at[pl.ds(rbase, ROWS_PER_TILE)],
                        cnt_out.at[pl.ds(rbase, ROWS_PER_TILE)])


@jax.jit
def _sc_aggregate(xlo, xhi, src, dst):
    mesh = plsc.VectorSubcoreMesh(core_axis_name="c", subcore_axis_name="s")
    f = pl.kernel(
        _sc_kernel,
        out_type=[
            jax.ShapeDtypeStruct((NC, N_PAD, DH), jnp.float32),
            jax.ShapeDtypeStruct((N_PAD, CNT_W), jnp.float32),
        ],
        mesh=mesh,
        scratch_types=[
            pltpu.VMEM((E_PER_S,), jnp.int32),
            pltpu.VMEM((E_PER_S,), jnp.int32),
            pltpu.VMEM((CHUNK, DH), jnp.float32),
            pltpu.VMEM((CHUNK, DH), jnp.float32),
            pltpu.VMEM((CHUNK, CNT_W), jnp.float32),
            pltpu.VMEM((128, DH), jnp.float32),
            pltpu.VMEM((ROWS_PER_TILE, CNT_W), jnp.float32),
            pltpu.VMEM_SHARED((N_PAD, DH), jnp.float32),
            pltpu.VMEM_SHARED((N_PAD, CNT_W), jnp.float32),
            pltpu.SemaphoreType.DMA,
            pltpu.SemaphoreType.DMA,
            pltpu.SemaphoreType.DMA,
            pltpu.SemaphoreType.DMA,
            pltpu.SemaphoreType.DMA,
        ],
        compiler_params=pltpu.CompilerParams(use_tc_tiling_on_sc=False),
    )
    return f(xlo, xhi, src, dst)


def _tc_finish_body(acc_ref, cnt_ref, w_ref, o_ref):
    s = jnp.concatenate([acc_ref[0], acc_ref[1]], axis=1)
    c = cnt_ref[:, 0]
    z = lax.dot_general(s, w_ref[...], (((1,), (1,)), ((), ())),
                        preferred_element_type=jnp.float32)
    o_ref[...] = z / jnp.maximum(c, 1.0)[:, None]


@jax.jit
def _tc_finish(acc, cnt, W):
    blk = 1024
    return pl.pallas_call(
        _tc_finish_body,
        grid=(N_PAD // blk,),
        in_specs=[
            pl.BlockSpec((NC, blk, DH), lambda i: (0, i, 0)),
            pl.BlockSpec((blk, CNT_W), lambda i: (i, 0)),
            pl.BlockSpec((D, D), lambda i: (0, 0)),
        ],
        out_specs=pl.BlockSpec((blk, D), lambda i: (i, 0)),
        out_shape=jax.ShapeDtypeStruct((N_PAD, D), jnp.float32),
    )(acc, cnt, W)


def kernel(x, edge_index, W):
    src = edge_index[0]
    dst = edge_index[1]
    xlo = x[:, :DH]
    xhi = x[:, DH:]
    acc, cnt = _sc_aggregate(xlo, xhi, src, dst)
    return _tc_finish(acc, cnt, W)[:N_NODES]


# speedup vs baseline: 1.0079x; 1.0079x over previous
import jax
import jax.numpy as jnp
from jax import lax
from jax.experimental import pallas as pl
from jax.experimental.pallas import tpu as pltpu
from jax.experimental.pallas import tpu_sc as plsc

N_NODES = 10000
N_PAD = 10240
N_EDGES = 320000
D = 128
DH = D // 2

NC = 2
NS = 16
E_PER_S = N_EDGES // NS
CHUNK = 80
N_CHUNKS = E_PER_S // CHUNK
ROWS_PER_TILE = N_PAD // NS
CNT_W = 16


def _sc_kernel(xlo_hbm, xhi_hbm, src_hbm, dst_hbm, acc_out, cnt_out,
               src_all, dst_all, rows0, rows1, zacc, hist, merge_buf, cnt_res,
               acc_sh, hist_stage, gsem0, gsem1, ssem0, ssem1):
    cid = lax.axis_index("c")
    sid = lax.axis_index("s")

    zero16 = jnp.zeros((16,), jnp.float32)
    one16 = jnp.ones((16,), jnp.float32)

    def init_zacc(i, carry):
        for c8 in range(DH // 16):
            zacc[i, pl.ds(c8 * 16, 16)] = zero16
        return carry

    lax.fori_loop(0, 128, init_zacc, 0)

    def init_hist(i, carry):
        hist[pl.ds(i * 16, 16)] = zero16
        return carry

    lax.fori_loop(0, N_PAD // 16, init_hist, 0)

    for b in range(ROWS_PER_TILE // 128):
        pltpu.sync_copy(zacc, acc_sh.at[pl.ds(sid * ROWS_PER_TILE + b * 128, 128)])

    plsc.subcore_barrier()

    ebase = sid * E_PER_S

    pltpu.sync_copy(src_hbm.at[pl.ds(ebase, E_PER_S)], src_all)
    pltpu.sync_copy(dst_hbm.at[pl.ds(ebase, E_PER_S)], dst_all)

    def gather(i, buf, sem):
        idx = src_all.at[pl.ds(i * CHUNK, CHUNK)]

        @pl.when(cid == 0)
        def _():
            pltpu.async_copy(xlo_hbm.at[idx], buf, sem)

        @pl.when(cid == 1)
        def _():
            pltpu.async_copy(xhi_hbm.at[idx], buf, sem)

    def gwait(buf, sem):
        pltpu.make_async_copy(xlo_hbm.at[pl.ds(0, CHUNK)], buf, sem).wait()

    def acc_scatter(i, buf, sem):
        didx = dst_all.at[pl.ds(i * CHUNK, CHUNK)]
        pltpu.async_copy(buf, acc_sh.at[didx], sem, add=True)

    def acc_swait(i, buf, sem):
        didx = dst_all.at[pl.ds(i * CHUNK, CHUNK)]
        pltpu.make_async_copy(buf, acc_sh.at[didx], sem).wait()

    gather(0, rows0, gsem0)

    def half(i, buf, gsem, ssem, obuf, ogsem, ossem):
        gwait(buf, gsem)

        @pl.when(i > 0)
        def _():
            acc_swait(i - 1, obuf, ossem)

        @pl.when(i + 1 < N_CHUNKS)
        def _():
            gather(i + 1, obuf, ogsem)

        acc_scatter(i, buf, ssem)

        @pl.when(cid == 0)
        def _():
            for g in range(CHUNK // 16):
                idx = dst_all[pl.ds(i * CHUNK + g * 16, 16)]
                plsc.addupdate_scatter(hist, [idx], one16)

    def two_chunks(j, carry):
        half(2 * j, rows0, gsem0, ssem0, rows1, gsem1, ssem1)
        half(2 * j + 1, rows1, gsem1, ssem1, rows0, gsem0, ssem0)
        return carry

    lax.fori_loop(0, N_CHUNKS // 2, two_chunks, 0)
    acc_swait(N_CHUNKS - 1, rows1, ssem1)

    @pl.when(cid == 0)
    def _():
        pltpu.sync_copy(hist, hist_stage.at[sid])

    plsc.subcore_barrier()

    rbase = sid * ROWS_PER_TILE
    pltpu.sync_copy(acc_sh.at[pl.ds(rbase, ROWS_PER_TILE)],
                    acc_out.at[cid, pl.ds(rbase, ROWS_PER_TILE)])

    @pl.when(cid == 0)
    def _():
        pltpu.sync_copy(hist_stage.at[:, pl.ds(rbase, ROWS_PER_TILE)], merge_buf)

        def merge_group(g, carry):
            s = merge_buf[0, pl.ds(g * 16, 16)]
            for r in range(1, NS):
                s = s + merge_buf[r, pl.ds(g * 16, 16)]
            cnt_res[pl.ds(g * 16, 16)] = s
            return carry

        lax.fori_loop(0, ROWS_PER_TILE // 16, merge_group, 0)
        pltpu.sync_copy(cnt_res, cnt_out.at[pl.ds(rbase, ROWS_PER_TILE)])


@jax.jit
def _sc_aggregate(xlo, xhi, src, dst):
    mesh = plsc.VectorSubcoreMesh(core_axis_name="c", subcore_axis_name="s")
    f = pl.kernel(
        _sc_kernel,
        out_type=[
            jax.ShapeDtypeStruct((NC, N_PAD, DH), jnp.float32),
            jax.ShapeDtypeStruct((N_PAD,), jnp.float32),
        ],
        mesh=mesh,
        scratch_types=[
            pltpu.VMEM((E_PER_S,), jnp.int32),
            pltpu.VMEM((E_PER_S,), jnp.int32),
            pltpu.VMEM((CHUNK, DH), jnp.float32),
            pltpu.VMEM((CHUNK, DH), jnp.float32),
            pltpu.VMEM((128, DH), jnp.float32),
            pltpu.VMEM((N_PAD,), jnp.float32),
            pltpu.VMEM((NS, ROWS_PER_TILE), jnp.float32),
            pltpu.VMEM((ROWS_PER_TILE,), jnp.float32),
            pltpu.VMEM_SHARED((N_PAD, DH), jnp.float32),
            pltpu.VMEM_SHARED((NS, N_PAD), jnp.float32),
            pltpu.SemaphoreType.DMA,
            pltpu.SemaphoreType.DMA,
            pltpu.SemaphoreType.DMA,
            pltpu.SemaphoreType.DMA,
        ],
        compiler_params=pltpu.CompilerParams(use_tc_tiling_on_sc=False,
                                             needs_layout_passes=False),
    )
    return f(xlo, xhi, src, dst)


def _tc_finish_body(acc_ref, cnt_ref, w_ref, o_ref):
    s = jnp.concatenate([acc_ref[0], acc_ref[1]], axis=1)
    c = cnt_ref[...]
    z = lax.dot_general(s, w_ref[...], (((1,), (1,)), ((), ())),
                        preferred_element_type=jnp.float32)
    o_ref[...] = z / jnp.maximum(c, 1.0)[:, None]


@jax.jit
def _tc_finish(acc, cnt, W):
    blk = 1024
    return pl.pallas_call(
        _tc_finish_body,
        grid=(N_PAD // blk,),
        in_specs=[
            pl.BlockSpec((NC, blk, DH), lambda i: (0, i, 0)),
            pl.BlockSpec((blk,), lambda i: (i,)),
            pl.BlockSpec((D, D), lambda i: (0, 0)),
        ],
        out_specs=pl.BlockSpec((blk, D), lambda i: (i, 0)),
        out_shape=jax.ShapeDtypeStruct((N_PAD, D), jnp.float32),
    )(acc, cnt, W)


def kernel(x, edge_index, W):
    src = edge_index[0]
    dst = edge_index[1]
    xlo = x[:, :DH]
    xhi = x[:, DH:]
    acc, cnt = _sc_aggregate(xlo, xhi, src, dst)
    return _tc_finish(acc, cnt, W)[:N_NODES]


# speedup vs baseline: 1.2433x; 1.2335x over previous
import jax
import jax.numpy as jnp
from jax import lax
from jax.experimental import pallas as pl
from jax.experimental.pallas import tpu as pltpu
from jax.experimental.pallas import tpu_sc as plsc

N_NODES = 10000
N_PAD = 10240
N_EDGES = 320000
D = 128
DH = D // 2

NC = 2
NS = 16
E_PER_S = N_EDGES // NS
CHUNK = 128
N_FULL = E_PER_S // CHUNK
TAIL = E_PER_S - N_FULL * CHUNK
ROWS_PER_TILE = N_PAD // NS
CNT_W = 16


def _sc_kernel(xlo_hbm, xhi_hbm, src_hbm, dst_hbm, acc_out, cnt_out,
               src_all, dst_all, rows0, rows1, zacc, hist, merge_buf, cnt_res,
               acc_sh, hist_stage, gsem0, gsem1, ssem0, ssem1):
    cid = lax.axis_index("c")
    sid = lax.axis_index("s")

    zero16 = jnp.zeros((16,), jnp.float32)
    one16 = jnp.ones((16,), jnp.float32)

    def init_zacc(i, carry):
        for c8 in range(DH // 16):
            zacc[i, pl.ds(c8 * 16, 16)] = zero16
        return carry

    lax.fori_loop(0, 128, init_zacc, 0)

    def init_hist(i, carry):
        hist[pl.ds(i * 16, 16)] = zero16
        return carry

    lax.fori_loop(0, N_PAD // 16, init_hist, 0)

    for b in range(ROWS_PER_TILE // 128):
        pltpu.sync_copy(zacc, acc_sh.at[pl.ds(sid * ROWS_PER_TILE + b * 128, 128)])

    plsc.subcore_barrier()

    ebase = sid * E_PER_S

    pltpu.sync_copy(src_hbm.at[pl.ds(ebase, E_PER_S)], src_all)
    pltpu.sync_copy(dst_hbm.at[pl.ds(ebase, E_PER_S)], dst_all)

    def gather(i, buf, sem):
        idx = src_all.at[pl.ds(i * CHUNK, CHUNK)]

        @pl.when(cid == 0)
        def _():
            pltpu.async_copy(xlo_hbm.at[idx], buf, sem)

        @pl.when(cid == 1)
        def _():
            pltpu.async_copy(xhi_hbm.at[idx], buf, sem)

    def gwait(buf, sem):
        pltpu.make_async_copy(xlo_hbm.at[pl.ds(0, CHUNK)], buf, sem).wait()

    def acc_scatter(i, buf, sem):
        didx = dst_all.at[pl.ds(i * CHUNK, CHUNK)]
        pltpu.async_copy(buf, acc_sh.at[didx], sem, add=True)

    def acc_swait(i, buf, sem):
        didx = dst_all.at[pl.ds(i * CHUNK, CHUNK)]
        pltpu.make_async_copy(buf, acc_sh.at[didx], sem).wait()

    gather(0, rows0, gsem0)

    def half(i, buf, gsem, ssem, obuf, ogsem, ossem):
        gwait(buf, gsem)

        @pl.when(i > 0)
        def _():
            acc_swait(i - 1, obuf, ossem)

        @pl.when(i + 1 < N_FULL)
        def _():
            gather(i + 1, obuf, ogsem)

        acc_scatter(i, buf, ssem)

        for g in range(CHUNK // 16):
            idx = dst_all[pl.ds(i * CHUNK + g * 16, 16)]
            plsc.addupdate_scatter(hist, [idx], one16)

    def two_chunks(j, carry):
        half(2 * j, rows0, gsem0, ssem0, rows1, gsem1, ssem1)
        half(2 * j + 1, rows1, gsem1, ssem1, rows0, gsem0, ssem0)
        return carry

    lax.fori_loop(0, N_FULL // 2, two_chunks, 0)
    acc_swait(N_FULL - 1, rows1, ssem1)

    tidx = src_all.at[pl.ds(N_FULL * CHUNK, TAIL)]
    tbuf = rows0.at[pl.ds(0, TAIL)]

    @pl.when(cid == 0)
    def _():
        pltpu.sync_copy(xlo_hbm.at[tidx], tbuf)

    @pl.when(cid == 1)
    def _():
        pltpu.sync_copy(xhi_hbm.at[tidx], tbuf)

    tdidx = dst_all.at[pl.ds(N_FULL * CHUNK, TAIL)]
    pltpu.sync_copy(tbuf, acc_sh.at[tdidx], add=True)

    for g in range(TAIL // 16):
        idx = dst_all[pl.ds(N_FULL * CHUNK + g * 16, 16)]
        plsc.addupdate_scatter(hist, [idx], one16)

    pltpu.sync_copy(hist.at[pl.ds(cid * (N_PAD // NC), N_PAD // NC)],
                    hist_stage.at[sid])

    plsc.subcore_barrier()

    rbase = sid * ROWS_PER_TILE
    pltpu.sync_copy(acc_sh.at[pl.ds(rbase, ROWS_PER_TILE)],
                    acc_out.at[cid, pl.ds(rbase, ROWS_PER_TILE)])

    mrows = ROWS_PER_TILE // NC
    pltpu.sync_copy(hist_stage.at[:, pl.ds(sid * mrows, mrows)], merge_buf)

    def merge_group(g, carry):
        s = merge_buf[0, pl.ds(g * 16, 16)]
        for r in range(1, NS):
            s = s + merge_buf[r, pl.ds(g * 16, 16)]
        cnt_res[pl.ds(g * 16, 16)] = s
        return carry

    lax.fori_loop(0, mrows // 16, merge_group, 0)
    pltpu.sync_copy(cnt_res,
                    cnt_out.at[pl.ds(cid * (N_PAD // NC) + sid * mrows, mrows)])


@jax.jit
def _sc_aggregate(xlo, xhi, src, dst):
    mesh = plsc.VectorSubcoreMesh(core_axis_name="c", subcore_axis_name="s")
    f = pl.kernel(
        _sc_kernel,
        out_type=[
            jax.ShapeDtypeStruct((NC, N_PAD, DH), jnp.float32),
            jax.ShapeDtypeStruct((N_PAD,), jnp.float32),
        ],
        mesh=mesh,
        scratch_types=[
            pltpu.VMEM((E_PER_S,), jnp.int32),
            pltpu.VMEM((E_PER_S,), jnp.int32),
            pltpu.VMEM((CHUNK, DH), jnp.float32),
            pltpu.VMEM((CHUNK, DH), jnp.float32),
            pltpu.VMEM((128, DH), jnp.float32),
            pltpu.VMEM((N_PAD,), jnp.float32),
            pltpu.VMEM((NS, ROWS_PER_TILE // NC), jnp.float32),
            pltpu.VMEM((ROWS_PER_TILE // NC,), jnp.float32),
            pltpu.VMEM_SHARED((N_PAD, DH), jnp.float32),
            pltpu.VMEM_SHARED((NS, N_PAD // NC), jnp.float32),
            pltpu.SemaphoreType.DMA,
            pltpu.SemaphoreType.DMA,
            pltpu.SemaphoreType.DMA,
            pltpu.SemaphoreType.DMA,
        ],
        compiler_params=pltpu.CompilerParams(use_tc_tiling_on_sc=False,
                                             needs_layout_passes=False),
    )
    return f(xlo, xhi, src, dst)


def _tc_finish_body(acc_ref, cnt_ref, w_ref, o_ref):
    s = jnp.concatenate([acc_ref[0], acc_ref[1]], axis=1)
    c = cnt_ref[...]
    z = lax.dot_general(s, w_ref[...], (((1,), (1,)), ((), ())),
                        preferred_element_type=jnp.float32)
    o_ref[...] = z / jnp.maximum(c, 1.0)[:, None]


@jax.jit
def _tc_finish(acc, cnt, W):
    blk = 1024
    return pl.pallas_call(
        _tc_finish_body,
        grid=(N_PAD // blk,),
        in_specs=[
            pl.BlockSpec((NC, blk, DH), lambda i: (0, i, 0)),
            pl.BlockSpec((blk,), lambda i: (i,)),
            pl.BlockSpec((D, D), lambda i: (0, 0)),
        ],
        out_specs=pl.BlockSpec((blk, D), lambda i: (i, 0)),
        out_shape=jax.ShapeDtypeStruct((N_PAD, D), jnp.float32),
    )(acc, cnt, W)


def kernel(x, edge_index, W):
    src = edge_index[0]
    dst = edge_index[1]
    xlo = x[:, :DH]
    xhi = x[:, DH:]
    acc, cnt = _sc_aggregate(xlo, xhi, src, dst)
    return _tc_finish(acc, cnt, W)[:N_NODES]


# speedup vs baseline: 1.7087x; 1.3743x over previous
import jax
import jax.numpy as jnp
from jax import lax
from jax.experimental import pallas as pl
from jax.experimental.pallas import tpu as pltpu
from jax.experimental.pallas import tpu_sc as plsc

N_NODES = 10000
N_PAD = 10240
N_EDGES = 320000
D = 128
DH = D // 2

NC = 2
NS = 16
E_PER_S = N_EDGES // NS
CHUNK = 128
N_FULL = E_PER_S // CHUNK
TAIL = E_PER_S - N_FULL * CHUNK
ROWS_PER_TILE = N_PAD // NS
CNT_W = 16


def _sc_kernel(xlo_hbm, xhi_hbm, src_hbm, dst_hbm, acc_out, cnt_out,
               src_all, dst_all, rows0, rows1, rows2, zacc, hist,
               acc_sh, gsem0, gsem1, gsem2, ssem0, ssem1, ssem2):
    cid = lax.axis_index("c")
    sid = lax.axis_index("s")

    zero16 = jnp.zeros((16,), jnp.float32)
    one16 = jnp.ones((16,), jnp.float32)

    def init_zacc(i, carry):
        for c8 in range(DH // 16):
            zacc[i, pl.ds(c8 * 16, 16)] = zero16
        return carry

    lax.fori_loop(0, 128, init_zacc, 0)

    def init_hist(i, carry):
        hist[pl.ds(i * 16, 16)] = zero16
        return carry

    lax.fori_loop(0, N_PAD // 16, init_hist, 0)

    for b in range(ROWS_PER_TILE // 128):
        pltpu.sync_copy(zacc, acc_sh.at[pl.ds(sid * ROWS_PER_TILE + b * 128, 128)])

    plsc.subcore_barrier()

    ebase = sid * E_PER_S

    pltpu.sync_copy(src_hbm.at[pl.ds(ebase, E_PER_S)], src_all)
    pltpu.sync_copy(dst_hbm.at[pl.ds(ebase, E_PER_S)], dst_all)

    def gather(i, buf, sem):
        idx = src_all.at[pl.ds(i * CHUNK, CHUNK)]

        @pl.when(cid == 0)
        def _():
            pltpu.async_copy(xlo_hbm.at[idx], buf, sem)

        @pl.when(cid == 1)
        def _():
            pltpu.async_copy(xhi_hbm.at[idx], buf, sem)

    def gwait(buf, sem):
        pltpu.make_async_copy(xlo_hbm.at[pl.ds(0, CHUNK)], buf, sem).wait()

    def acc_scatter(i, buf, sem):
        didx = dst_all.at[pl.ds(i * CHUNK, CHUNK)]
        pltpu.async_copy(buf, acc_sh.at[didx], sem, add=True)

    def acc_swait(i, buf, sem):
        didx = dst_all.at[pl.ds(i * CHUNK, CHUNK)]
        pltpu.make_async_copy(buf, acc_sh.at[didx], sem).wait()

    BUFS = [(rows0, gsem0, ssem0), (rows1, gsem1, ssem1),
            (rows2, gsem2, ssem2)]
    NBUF = len(BUFS)

    for k in range(NBUF - 1):
        gather(k, BUFS[k][0], BUFS[k][1])

    def stage(i, buf, gsem, ssem, nbuf, ngsem, nssem):
        gwait(buf, gsem)

        @pl.when(i > 0)
        def _():
            acc_swait(i - 1, nbuf, nssem)

        @pl.when(i + NBUF - 1 < N_FULL)
        def _():
            gather(i + NBUF - 1, nbuf, ngsem)

        acc_scatter(i, buf, ssem)

        for g in range(CHUNK // 16):
            idx = dst_all[pl.ds(i * CHUNK + g * 16, 16)]
            plsc.addupdate_scatter(hist, [idx], one16)

    def nbuf_chunks(j, carry):
        for k in range(NBUF):
            buf, gsem, ssem = BUFS[k]
            nbuf, ngsem, nssem = BUFS[(k + NBUF - 1) % NBUF]
            stage(NBUF * j + k, buf, gsem, ssem, nbuf, ngsem, nssem)
        return carry

    lax.fori_loop(0, N_FULL // NBUF, nbuf_chunks, 0)
    acc_swait(N_FULL - 1, rows2, ssem2)

    tidx = src_all.at[pl.ds(N_FULL * CHUNK, TAIL)]
    tbuf = rows0.at[pl.ds(0, TAIL)]

    @pl.when(cid == 0)
    def _():
        pltpu.sync_copy(xlo_hbm.at[tidx], tbuf)

    @pl.when(cid == 1)
    def _():
        pltpu.sync_copy(xhi_hbm.at[tidx], tbuf)

    tdidx = dst_all.at[pl.ds(N_FULL * CHUNK, TAIL)]
    pltpu.sync_copy(tbuf, acc_sh.at[tdidx], add=True)

    for g in range(TAIL // 16):
        idx = dst_all[pl.ds(N_FULL * CHUNK + g * 16, 16)]
        plsc.addupdate_scatter(hist, [idx], one16)

    @pl.when(cid == 0)
    def _():
        pltpu.sync_copy(hist, cnt_out.at[sid])

    plsc.subcore_barrier()

    rbase = sid * ROWS_PER_TILE
    pltpu.sync_copy(acc_sh.at[pl.ds(rbase, ROWS_PER_TILE)],
                    acc_out.at[cid, pl.ds(rbase, ROWS_PER_TILE)])


@jax.jit
def _sc_aggregate(xlo, xhi, src, dst):
    mesh = plsc.VectorSubcoreMesh(core_axis_name="c", subcore_axis_name="s")
    f = pl.kernel(
        _sc_kernel,
        out_type=[
            jax.ShapeDtypeStruct((NC, N_PAD, DH), jnp.float32),
            jax.ShapeDtypeStruct((NS, N_PAD), jnp.float32),
        ],
        mesh=mesh,
        scratch_types=[
            pltpu.VMEM((E_PER_S,), jnp.int32),
            pltpu.VMEM((E_PER_S,), jnp.int32),
            pltpu.VMEM((CHUNK, DH), jnp.float32),
            pltpu.VMEM((CHUNK, DH), jnp.float32),
            pltpu.VMEM((CHUNK, DH), jnp.float32),
            pltpu.VMEM((128, DH), jnp.float32),
            pltpu.VMEM((N_PAD,), jnp.float32),
            pltpu.VMEM_SHARED((N_PAD, DH), jnp.float32),
            pltpu.SemaphoreType.DMA,
            pltpu.SemaphoreType.DMA,
            pltpu.SemaphoreType.DMA,
            pltpu.SemaphoreType.DMA,
            pltpu.SemaphoreType.DMA,
            pltpu.SemaphoreType.DMA,
        ],
        compiler_params=pltpu.CompilerParams(use_tc_tiling_on_sc=False,
                                             needs_layout_passes=False),
    )
    return f(xlo, xhi, src, dst)


def _tc_finish_body(acc_ref, cnt_ref, w_ref, o_ref):
    s = jnp.concatenate([acc_ref[0], acc_ref[1]], axis=1)
    c = jnp.sum(cnt_ref[...], axis=0)
    z = lax.dot_general(s, w_ref[...], (((1,), (1,)), ((), ())),
                        preferred_element_type=jnp.float32)
    o_ref[...] = z / jnp.maximum(c, 1.0)[:, None]


@jax.jit
def _tc_finish(acc, cnt, W):
    blk = 1024
    return pl.pallas_call(
        _tc_finish_body,
        grid=(N_PAD // blk,),
        in_specs=[
            pl.BlockSpec((NC, blk, DH), lambda i: (0, i, 0)),
            pl.BlockSpec((NS, blk), lambda i: (0, i)),
            pl.BlockSpec((D, D), lambda i: (0, 0)),
        ],
        out_specs=pl.BlockSpec((blk, D), lambda i: (i, 0)),
        out_shape=jax.ShapeDtypeStruct((N_PAD, D), jnp.float32),
    )(acc, cnt, W)


def kernel(x, edge_index, W):
    src = edge_index[0]
    dst = edge_index[1]
    xlo = x[:, :DH]
    xhi = x[:, DH:]
    acc, cnt = _sc_aggregate(xlo, xhi, src, dst)
    return _tc_finish(acc, cnt, W)[:N_NODES]


# speedup vs baseline: 1.8124x; 1.0607x over previous
import jax
import jax.numpy as jnp
from jax import lax
from jax.experimental import pallas as pl
from jax.experimental.pallas import tpu as pltpu
from jax.experimental.pallas import tpu_sc as plsc

N_NODES = 10000
N_PAD = 10240
N_EDGES = 320000
D = 128
DH = D // 2

NC = 2
NS = 16
E_PER_S = N_EDGES // NS
CHUNK = 128
N_FULL = E_PER_S // CHUNK
TAIL = E_PER_S - N_FULL * CHUNK
ROWS_PER_TILE = N_PAD // NS
CNT_W = 16


def _sc_kernel(xlo_hbm, xhi_hbm, ei_hbm, acc_out, cnt_out,
               src_all, dst_all, rows0, rows1, rows2, zacc, hist,
               acc_sh, gsem0, gsem1, gsem2, ssem0, ssem1, ssem2):
    cid = lax.axis_index("c")
    sid = lax.axis_index("s")

    zero16 = jnp.zeros((16,), jnp.float32)
    one16 = jnp.ones((16,), jnp.float32)

    def init_zacc(i, carry):
        for c8 in range(DH // 16):
            zacc[i, pl.ds(c8 * 16, 16)] = zero16
        return carry

    lax.fori_loop(0, 128, init_zacc, 0)

    def init_hist(i, carry):
        hist[pl.ds(i * 16, 16)] = zero16
        return carry

    lax.fori_loop(0, N_PAD // 16, init_hist, 0)

    for b in range(ROWS_PER_TILE // 128):
        pltpu.sync_copy(zacc, acc_sh.at[pl.ds(sid * ROWS_PER_TILE + b * 128, 128)])

    plsc.subcore_barrier()

    ebase = sid * E_PER_S

    pltpu.sync_copy(ei_hbm.at[0, pl.ds(ebase, E_PER_S)], src_all)
    pltpu.sync_copy(ei_hbm.at[1, pl.ds(ebase, E_PER_S)], dst_all)

    def gather(i, buf, sem):
        idx = src_all.at[pl.ds(i * CHUNK, CHUNK)]

        @pl.when(cid == 0)
        def _():
            pltpu.async_copy(xlo_hbm.at[idx], buf, sem)

        @pl.when(cid == 1)
        def _():
            pltpu.async_copy(xhi_hbm.at[idx], buf, sem)

    def gwait(buf, sem):
        pltpu.make_async_copy(xlo_hbm.at[pl.ds(0, CHUNK)], buf, sem).wait()

    def acc_scatter(i, buf, sem):
        didx = dst_all.at[pl.ds(i * CHUNK, CHUNK)]
        pltpu.async_copy(buf, acc_sh.at[didx], sem, add=True)

    def acc_swait(i, buf, sem):
        didx = dst_all.at[pl.ds(i * CHUNK, CHUNK)]
        pltpu.make_async_copy(buf, acc_sh.at[didx], sem).wait()

    BUFS = [(rows0, gsem0, ssem0), (rows1, gsem1, ssem1),
            (rows2, gsem2, ssem2)]
    NBUF = len(BUFS)

    for k in range(NBUF - 1):
        gather(k, BUFS[k][0], BUFS[k][1])

    def stage(i, buf, gsem, ssem, nbuf, ngsem, nssem):
        gwait(buf, gsem)

        @pl.when(i > 0)
        def _():
            acc_swait(i - 1, nbuf, nssem)

        @pl.when(i + NBUF - 1 < N_FULL)
        def _():
            gather(i + NBUF - 1, nbuf, ngsem)

        acc_scatter(i, buf, ssem)

        for g in range(CHUNK // 16):
            idx = dst_all[pl.ds(i * CHUNK + g * 16, 16)]
            plsc.addupdate_scatter(hist, [idx], one16)

    def nbuf_chunks(j, carry):
        for k in range(NBUF):
            buf, gsem, ssem = BUFS[k]
            nbuf, ngsem, nssem = BUFS[(k + NBUF - 1) % NBUF]
            stage(NBUF * j + k, buf, gsem, ssem, nbuf, ngsem, nssem)
        return carry

    lax.fori_loop(0, N_FULL // NBUF, nbuf_chunks, 0)
    acc_swait(N_FULL - 1, rows2, ssem2)

    tidx = src_all.at[pl.ds(N_FULL * CHUNK, TAIL)]
    tbuf = rows0.at[pl.ds(0, TAIL)]

    @pl.when(cid == 0)
    def _():
        pltpu.sync_copy(xlo_hbm.at[tidx], tbuf)

    @pl.when(cid == 1)
    def _():
        pltpu.sync_copy(xhi_hbm.at[tidx], tbuf)

    tdidx = dst_all.at[pl.ds(N_FULL * CHUNK, TAIL)]
    pltpu.sync_copy(tbuf, acc_sh.at[tdidx], add=True)

    for g in range(TAIL // 16):
        idx = dst_all[pl.ds(N_FULL * CHUNK + g * 16, 16)]
        plsc.addupdate_scatter(hist, [idx], one16)

    @pl.when(cid == 0)
    def _():
        pltpu.sync_copy(hist, cnt_out.at[sid])

    plsc.subcore_barrier()

    rbase = sid * ROWS_PER_TILE
    pltpu.sync_copy(acc_sh.at[pl.ds(rbase, ROWS_PER_TILE)],
                    acc_out.at[cid, pl.ds(rbase, ROWS_PER_TILE)])


@jax.jit
def _sc_aggregate(xlo, xhi, edge_index):
    mesh = plsc.VectorSubcoreMesh(core_axis_name="c", subcore_axis_name="s")
    f = pl.kernel(
        _sc_kernel,
        out_type=[
            jax.ShapeDtypeStruct((NC, N_PAD, DH), jnp.float32),
            jax.ShapeDtypeStruct((NS, N_PAD), jnp.float32),
        ],
        mesh=mesh,
        scratch_types=[
            pltpu.VMEM((E_PER_S,), jnp.int32),
            pltpu.VMEM((E_PER_S,), jnp.int32),
            pltpu.VMEM((CHUNK, DH), jnp.float32),
            pltpu.VMEM((CHUNK, DH), jnp.float32),
            pltpu.VMEM((CHUNK, DH), jnp.float32),
            pltpu.VMEM((128, DH), jnp.float32),
            pltpu.VMEM((N_PAD,), jnp.float32),
            pltpu.VMEM_SHARED((N_PAD, DH), jnp.float32),
            pltpu.SemaphoreType.DMA,
            pltpu.SemaphoreType.DMA,
            pltpu.SemaphoreType.DMA,
            pltpu.SemaphoreType.DMA,
            pltpu.SemaphoreType.DMA,
            pltpu.SemaphoreType.DMA,
        ],
        compiler_params=pltpu.CompilerParams(use_tc_tiling_on_sc=False,
                                             needs_layout_passes=False),
    )
    return f(xlo, xhi, edge_index)


def _tc_finish_body(acc_ref, cnt_ref, w_ref, o_ref):
    s = jnp.concatenate([acc_ref[0], acc_ref[1]], axis=1)
    c = jnp.sum(cnt_ref[...], axis=0)
    z = lax.dot_general(s, w_ref[...], (((1,), (1,)), ((), ())),
                        preferred_element_type=jnp.float32)
    o_ref[...] = z / jnp.maximum(c, 1.0)[:, None]


@jax.jit
def _tc_finish(acc, cnt, W):
    blk = 1024
    return pl.pallas_call(
        _tc_finish_body,
        grid=(N_PAD // blk,),
        in_specs=[
            pl.BlockSpec((NC, blk, DH), lambda i: (0, i, 0)),
            pl.BlockSpec((NS, blk), lambda i: (0, i)),
            pl.BlockSpec((D, D), lambda i: (0, 0)),
        ],
        out_specs=pl.BlockSpec((blk, D), lambda i: (i, 0)),
        out_shape=jax.ShapeDtypeStruct((N_PAD, D), jnp.float32),
    )(acc, cnt, W)


def kernel(x, edge_index, W):
    acc, cnt = _sc_aggregate(x[:, :DH], x[:, DH:], edge_index)
    return _tc_finish(acc, cnt, W)[:N_NODES]


# speedup vs baseline: 1.9340x; 1.0671x over previous
import jax
import jax.numpy as jnp
from jax import lax
from jax.experimental import pallas as pl
from jax.experimental.pallas import tpu as pltpu
from jax.experimental.pallas import tpu_sc as plsc

N_NODES = 10000
N_PAD = 10240
N_EDGES = 320000
D = 128
DH = D // 2

NC = 2
NS = 16
E_PER_S = N_EDGES // NS
CHUNK = 128
N_FULL = E_PER_S // CHUNK
TAIL = E_PER_S - N_FULL * CHUNK
ROWS_PER_TILE = N_PAD // NS
CNT_W = 16


def _sc_kernel(xv_hbm, ei_hbm, acc_out, cnt_out,
               src_all, dst_all, idx0, idx1, idx2, rows0, rows1, rows2,
               zacc, hist, acc_sh, gsem0, gsem1, gsem2, ssem0, ssem1, ssem2):
    cid = lax.axis_index("c")
    sid = lax.axis_index("s")

    zero16 = jnp.zeros((16,), jnp.float32)
    one16 = jnp.ones((16,), jnp.float32)

    def init_zacc(i, carry):
        for c8 in range(DH // 16):
            zacc[i, pl.ds(c8 * 16, 16)] = zero16
        return carry

    lax.fori_loop(0, 128, init_zacc, 0)

    def init_hist(i, carry):
        hist[pl.ds(i * 16, 16)] = zero16
        return carry

    lax.fori_loop(0, N_PAD // 16, init_hist, 0)

    for b in range(ROWS_PER_TILE // 128):
        pltpu.sync_copy(zacc, acc_sh.at[pl.ds(sid * ROWS_PER_TILE + b * 128, 128)])

    plsc.subcore_barrier()

    ebase = sid * E_PER_S

    pltpu.sync_copy(ei_hbm.at[0, pl.ds(ebase, E_PER_S)], src_all)
    pltpu.sync_copy(ei_hbm.at[1, pl.ds(ebase, E_PER_S)], dst_all)

    def gather(i, buf, ibuf, sem):
        for g in range(CHUNK // 16):
            v = src_all[pl.ds(i * CHUNK + g * 16, 16)]
            ibuf[pl.ds(g * 16, 16)] = v + v + cid
        pltpu.async_copy(xv_hbm.at[ibuf], buf, sem)

    def gwait(buf, sem):
        pltpu.make_async_copy(xv_hbm.at[pl.ds(0, CHUNK)], buf, sem).wait()

    def acc_scatter(i, buf, sem):
        didx = dst_all.at[pl.ds(i * CHUNK, CHUNK)]
        pltpu.async_copy(buf, acc_sh.at[didx], sem, add=True)

    def acc_swait(i, buf, sem):
        didx = dst_all.at[pl.ds(i * CHUNK, CHUNK)]
        pltpu.make_async_copy(buf, acc_sh.at[didx], sem).wait()

    BUFS = [(rows0, idx0, gsem0, ssem0), (rows1, idx1, gsem1, ssem1),
            (rows2, idx2, gsem2, ssem2)]
    NBUF = len(BUFS)

    for k in range(NBUF - 1):
        gather(k, BUFS[k][0], BUFS[k][1], BUFS[k][2])

    def stage(i, buf, gsem, ssem, nbuf, nibuf, ngsem, nssem):
        gwait(buf, gsem)

        @pl.when(i > 0)
        def _():
            acc_swait(i - 1, nbuf, nssem)

        @pl.when(i + NBUF - 1 < N_FULL)
        def _():
            gather(i + NBUF - 1, nbuf, nibuf, ngsem)

        acc_scatter(i, buf, ssem)

        for g in range(CHUNK // 16):
            idx = dst_all[pl.ds(i * CHUNK + g * 16, 16)]
            plsc.addupdate_scatter(hist, [idx], one16)

    def nbuf_chunks(j, carry):
        for k in range(NBUF):
            buf, _, gsem, ssem = BUFS[k]
            nbuf, nibuf, ngsem, nssem = BUFS[(k + NBUF - 1) % NBUF]
            stage(NBUF * j + k, buf, gsem, ssem, nbuf, nibuf, ngsem, nssem)
        return carry

    lax.fori_loop(0, N_FULL // NBUF, nbuf_chunks, 0)
    acc_swait(N_FULL - 1, rows2, ssem2)

    for g in range(TAIL // 16):
        v = src_all[pl.ds(N_FULL * CHUNK + g * 16, 16)]
        idx0[pl.ds(g * 16, 16)] = v + v + cid
    tidx = idx0.at[pl.ds(0, TAIL)]
    tbuf = rows0.at[pl.ds(0, TAIL)]
    pltpu.sync_copy(xv_hbm.at[tidx], tbuf)

    tdidx = dst_all.at[pl.ds(N_FULL * CHUNK, TAIL)]
    pltpu.sync_copy(tbuf, acc_sh.at[tdidx], add=True)

    for g in range(TAIL // 16):
        idx = dst_all[pl.ds(N_FULL * CHUNK + g * 16, 16)]
        plsc.addupdate_scatter(hist, [idx], one16)

    @pl.when(cid == 0)
    def _():
        pltpu.sync_copy(hist, cnt_out.at[sid])

    plsc.subcore_barrier()

    rbase = sid * ROWS_PER_TILE
    pltpu.sync_copy(acc_sh.at[pl.ds(rbase, ROWS_PER_TILE)],
                    acc_out.at[cid, pl.ds(rbase, ROWS_PER_TILE)])


@jax.jit
def _sc_aggregate(xv, edge_index):
    mesh = plsc.VectorSubcoreMesh(core_axis_name="c", subcore_axis_name="s")
    f = pl.kernel(
        _sc_kernel,
        out_type=[
            jax.ShapeDtypeStruct((NC, N_PAD, DH), jnp.float32),
            jax.ShapeDtypeStruct((NS, N_PAD), jnp.float32),
        ],
        mesh=mesh,
        scratch_types=[
            pltpu.VMEM((E_PER_S,), jnp.int32),
            pltpu.VMEM((E_PER_S,), jnp.int32),
            pltpu.VMEM((CHUNK,), jnp.int32),
            pltpu.VMEM((CHUNK,), jnp.int32),
            pltpu.VMEM((CHUNK,), jnp.int32),
            pltpu.VMEM((CHUNK, DH), jnp.float32),
            pltpu.VMEM((CHUNK, DH), jnp.float32),
            pltpu.VMEM((CHUNK, DH), jnp.float32),
            pltpu.VMEM((128, DH), jnp.float32),
            pltpu.VMEM((N_PAD,), jnp.float32),
            pltpu.VMEM_SHARED((N_PAD, DH), jnp.float32),
            pltpu.SemaphoreType.DMA,
            pltpu.SemaphoreType.DMA,
            pltpu.SemaphoreType.DMA,
            pltpu.SemaphoreType.DMA,
            pltpu.SemaphoreType.DMA,
            pltpu.SemaphoreType.DMA,
        ],
        compiler_params=pltpu.CompilerParams(use_tc_tiling_on_sc=False,
                                             needs_layout_passes=False),
    )
    return f(xv, edge_index)


def _tc_finish_body(acc_ref, cnt_ref, w_ref, o_ref):
    s = jnp.concatenate([acc_ref[0], acc_ref[1]], axis=1)
    c = jnp.sum(cnt_ref[...], axis=0)
    z = lax.dot_general(s, w_ref[...], (((1,), (1,)), ((), ())),
                        preferred_element_type=jnp.float32)
    o_ref[...] = z / jnp.maximum(c, 1.0)[:, None]


@jax.jit
def _tc_finish(acc, cnt, W):
    blk = 1024
    return pl.pallas_call(
        _tc_finish_body,
        grid=(N_PAD // blk,),
        in_specs=[
            pl.BlockSpec((NC, blk, DH), lambda i: (0, i, 0)),
            pl.BlockSpec((NS, blk), lambda i: (0, i)),
            pl.BlockSpec((D, D), lambda i: (0, 0)),
        ],
        out_specs=pl.BlockSpec((blk, D), lambda i: (i, 0)),
        out_shape=jax.ShapeDtypeStruct((N_PAD, D), jnp.float32),
    )(acc, cnt, W)


def kernel(x, edge_index, W):
    acc, cnt = _sc_aggregate(x.reshape(2 * N_NODES, DH), edge_index)
    return _tc_finish(acc, cnt, W)[:N_NODES]


# speedup vs baseline: 1.9955x; 1.0318x over previous
import jax
import jax.numpy as jnp
from jax import lax
from jax.experimental import pallas as pl
from jax.experimental.pallas import tpu as pltpu
from jax.experimental.pallas import tpu_sc as plsc

N_NODES = 10000
N_PAD = 10240
N_EDGES = 320000
D = 128
DH = D // 2

NC = 2
NS = 16
E_PER_S = N_EDGES // NS
CHUNK = 128
N_FULL = E_PER_S // CHUNK
TAIL = E_PER_S - N_FULL * CHUNK
ROWS_PER_TILE = N_PAD // NS
CNT_W = 16


def _sc_kernel(xv_hbm, ei_hbm, acc_out, cnt_out,
               src_all, dst_all, idx0, idx1, idx2, rows0, rows1, rows2,
               zacc, hist, acc_sh, gsem0, gsem1, gsem2, ssem0, ssem1, ssem2):
    cid = lax.axis_index("c")
    sid = lax.axis_index("s")

    zero16 = jnp.zeros((16,), jnp.float32)
    one16 = jnp.ones((16,), jnp.float32)

    def init_zacc(i, carry):
        for c8 in range(DH // 16):
            zacc[i, pl.ds(c8 * 16, 16)] = zero16
        return carry

    lax.fori_loop(0, 128, init_zacc, 0)

    ebase = sid * E_PER_S

    for b in range(ROWS_PER_TILE // 128):
        pltpu.async_copy(zacc,
                         acc_sh.at[pl.ds(sid * ROWS_PER_TILE + b * 128, 128)],
                         ssem0)
    pltpu.async_copy(ei_hbm.at[0, pl.ds(ebase, E_PER_S)], src_all, gsem0)
    pltpu.async_copy(ei_hbm.at[1, pl.ds(ebase, E_PER_S)], dst_all, gsem1)

    def init_hist(i, carry):
        hist[pl.ds(i * 16, 16)] = zero16
        return carry

    lax.fori_loop(0, N_PAD // 16, init_hist, 0)

    for b in range(ROWS_PER_TILE // 128):
        pltpu.make_async_copy(
            zacc, acc_sh.at[pl.ds(sid * ROWS_PER_TILE + b * 128, 128)],
            ssem0).wait()
    pltpu.make_async_copy(ei_hbm.at[0, pl.ds(ebase, E_PER_S)], src_all,
                          gsem0).wait()
    pltpu.make_async_copy(ei_hbm.at[1, pl.ds(ebase, E_PER_S)], dst_all,
                          gsem1).wait()

    plsc.subcore_barrier()

    def gather(i, buf, ibuf, sem):
        for g in range(CHUNK // 16):
            v = src_all[pl.ds(i * CHUNK + g * 16, 16)]
            ibuf[pl.ds(g * 16, 16)] = v + v + cid
        pltpu.async_copy(xv_hbm.at[ibuf], buf, sem)

    def gwait(buf, sem):
        pltpu.make_async_copy(xv_hbm.at[pl.ds(0, CHUNK)], buf, sem).wait()

    def acc_scatter(i, buf, sem):
        didx = dst_all.at[pl.ds(i * CHUNK, CHUNK)]
        pltpu.async_copy(buf, acc_sh.at[didx], sem, add=True)

    def acc_swait(i, buf, sem):
        didx = dst_all.at[pl.ds(i * CHUNK, CHUNK)]
        pltpu.make_async_copy(buf, acc_sh.at[didx], sem).wait()

    BUFS = [(rows0, idx0, gsem0, ssem0), (rows1, idx1, gsem1, ssem1),
            (rows2, idx2, gsem2, ssem2)]
    NBUF = len(BUFS)

    for k in range(NBUF - 1):
        gather(k, BUFS[k][0], BUFS[k][1], BUFS[k][2])

    def stage(i, buf, gsem, ssem, nbuf, nibuf, ngsem, nssem):
        gwait(buf, gsem)

        @pl.when(i > 0)
        def _():
            acc_swait(i - 1, nbuf, nssem)

        @pl.when(i + NBUF - 1 < N_FULL)
        def _():
            gather(i + NBUF - 1, nbuf, nibuf, ngsem)

        acc_scatter(i, buf, ssem)

        for g in range(CHUNK // 16):
            idx = dst_all[pl.ds(i * CHUNK + g * 16, 16)]
            plsc.addupdate_scatter(hist, [idx], one16)

    def nbuf_chunks(j, carry):
        for k in range(NBUF):
            buf, _, gsem, ssem = BUFS[k]
            nbuf, nibuf, ngsem, nssem = BUFS[(k + NBUF - 1) % NBUF]
            stage(NBUF * j + k, buf, gsem, ssem, nbuf, nibuf, ngsem, nssem)
        return carry

    lax.fori_loop(0, N_FULL // NBUF, nbuf_chunks, 0)
    acc_swait(N_FULL - 1, rows2, ssem2)

    for g in range(TAIL // 16):
        v = src_all[pl.ds(N_FULL * CHUNK + g * 16, 16)]
        idx0[pl.ds(g * 16, 16)] = v + v + cid
    tidx = idx0.at[pl.ds(0, TAIL)]
    tbuf = rows0.at[pl.ds(0, TAIL)]
    pltpu.sync_copy(xv_hbm.at[tidx], tbuf)

    tdidx = dst_all.at[pl.ds(N_FULL * CHUNK, TAIL)]
    pltpu.sync_copy(tbuf, acc_sh.at[tdidx], add=True)

    for g in range(TAIL // 16):
        idx = dst_all[pl.ds(N_FULL * CHUNK + g * 16, 16)]
        plsc.addupdate_scatter(hist, [idx], one16)

    @pl.when(cid == 0)
    def _():
        pltpu.sync_copy(hist, cnt_out.at[sid])

    plsc.subcore_barrier()

    rbase = sid * ROWS_PER_TILE
    pltpu.sync_copy(acc_sh.at[pl.ds(rbase, ROWS_PER_TILE)],
                    acc_out.at[cid, pl.ds(rbase, ROWS_PER_TILE)])


@jax.jit
def _sc_aggregate(xv, edge_index):
    mesh = plsc.VectorSubcoreMesh(core_axis_name="c", subcore_axis_name="s")
    f = pl.kernel(
        _sc_kernel,
        out_type=[
            jax.ShapeDtypeStruct((NC, N_PAD, DH), jnp.float32),
            jax.ShapeDtypeStruct((NS, N_PAD), jnp.float32),
        ],
        mesh=mesh,
        scratch_types=[
            pltpu.VMEM((E_PER_S,), jnp.int32),
            pltpu.VMEM((E_PER_S,), jnp.int32),
            pltpu.VMEM((CHUNK,), jnp.int32),
            pltpu.VMEM((CHUNK,), jnp.int32),
            pltpu.VMEM((CHUNK,), jnp.int32),
            pltpu.VMEM((CHUNK, DH), jnp.float32),
            pltpu.VMEM((CHUNK, DH), jnp.float32),
            pltpu.VMEM((CHUNK, DH), jnp.float32),
            pltpu.VMEM((128, DH), jnp.float32),
            pltpu.VMEM((N_PAD,), jnp.float32),
            pltpu.VMEM_SHARED((N_PAD, DH), jnp.float32),
            pltpu.SemaphoreType.DMA,
            pltpu.SemaphoreType.DMA,
            pltpu.SemaphoreType.DMA,
            pltpu.SemaphoreType.DMA,
            pltpu.SemaphoreType.DMA,
            pltpu.SemaphoreType.DMA,
        ],
        compiler_params=pltpu.CompilerParams(use_tc_tiling_on_sc=False,
                                             needs_layout_passes=False),
    )
    return f(xv, edge_index)


def _tc_finish_body(acc_ref, cnt_ref, w_ref, o_ref):
    s = jnp.concatenate([acc_ref[0], acc_ref[1]], axis=1)
    c = jnp.sum(cnt_ref[...], axis=0)
    z = lax.dot_general(s, w_ref[...], (((1,), (1,)), ((), ())),
                        preferred_element_type=jnp.float32)
    o_ref[...] = z / jnp.maximum(c, 1.0)[:, None]


@jax.jit
def _tc_finish(acc, cnt, W):
    blk = 1024
    return pl.pallas_call(
        _tc_finish_body,
        grid=(N_PAD // blk,),
        in_specs=[
            pl.BlockSpec((NC, blk, DH), lambda i: (0, i, 0)),
            pl.BlockSpec((NS, blk), lambda i: (0, i)),
            pl.BlockSpec((D, D), lambda i: (0, 0)),
        ],
        out_specs=pl.BlockSpec((blk, D), lambda i: (i, 0)),
        out_shape=jax.ShapeDtypeStruct((N_PAD, D), jnp.float32),
    )(acc, cnt, W)


def kernel(x, edge_index, W):
    acc, cnt = _sc_aggregate(x.reshape(2 * N_NODES, DH), edge_index)
    return _tc_finish(acc, cnt, W)[:N_NODES]


# speedup vs baseline: 1.9989x; 1.0017x over previous
import jax
import jax.numpy as jnp
from jax import lax
from jax.experimental import pallas as pl
from jax.experimental.pallas import tpu as pltpu
from jax.experimental.pallas import tpu_sc as plsc

N_NODES = 10000
N_PAD = 10240
N_EDGES = 320000
D = 128
DH = D // 2

NC = 2
NS = 16
E_PER_S = N_EDGES // NS
CHUNK = 128
N_FULL = E_PER_S // CHUNK
TAIL = E_PER_S - N_FULL * CHUNK
ROWS_PER_TILE = N_PAD // NS


def _sc_kernel(xv_hbm, ei_hbm, acc_out, cnt_out,
               src_all, dst_all, idx0, idx1, idx2, rows0, rows1, rows2,
               zacc, hist, acc_sh, gsem0, gsem1, gsem2, ssem0, ssem1, ssem2):
    cid = lax.axis_index("c")
    sid = lax.axis_index("s")

    zero16 = jnp.zeros((16,), jnp.float32)
    one16 = jnp.ones((16,), jnp.float32)

    def init_zacc(i, carry):
        for c8 in range(DH // 16):
            zacc[i, pl.ds(c8 * 16, 16)] = zero16
        return carry

    lax.fori_loop(0, 128, init_zacc, 0)

    ebase = sid * E_PER_S

    for b in range(ROWS_PER_TILE // 128):
        pltpu.async_copy(zacc,
                         acc_sh.at[pl.ds(sid * ROWS_PER_TILE + b * 128, 128)],
                         ssem0)
    pltpu.async_copy(ei_hbm.at[0, pl.ds(ebase, E_PER_S)], src_all, gsem0)
    pltpu.async_copy(ei_hbm.at[1, pl.ds(ebase, E_PER_S)], dst_all, gsem1)

    def init_hist(i, carry):
        hist[pl.ds(i * 16, 16)] = zero16
        return carry

    lax.fori_loop(0, N_PAD // 16, init_hist, 0)

    for b in range(ROWS_PER_TILE // 128):
        pltpu.make_async_copy(
            zacc, acc_sh.at[pl.ds(sid * ROWS_PER_TILE + b * 128, 128)],
            ssem0).wait()
    pltpu.make_async_copy(ei_hbm.at[0, pl.ds(ebase, E_PER_S)], src_all,
                          gsem0).wait()
    pltpu.make_async_copy(ei_hbm.at[1, pl.ds(ebase, E_PER_S)], dst_all,
                          gsem1).wait()

    plsc.subcore_barrier()

    def gather(i, buf, ibuf, sem):
        for g in range(CHUNK // 16):
            v = src_all[pl.ds(i * CHUNK + g * 16, 16)]
            ibuf[pl.ds(g * 16, 16)] = v + v + cid
        pltpu.async_copy(xv_hbm.at[ibuf], buf, sem)

    def gwait(buf, sem):
        pltpu.make_async_copy(xv_hbm.at[pl.ds(0, CHUNK)], buf, sem).wait()

    def acc_scatter(i, buf, sem):
        didx = dst_all.at[pl.ds(i * CHUNK, CHUNK)]
        pltpu.async_copy(buf, acc_sh.at[didx], sem, add=True)

    def acc_swait(i, buf, sem):
        didx = dst_all.at[pl.ds(i * CHUNK, CHUNK)]
        pltpu.make_async_copy(buf, acc_sh.at[didx], sem).wait()

    BUFS = [(rows0, idx0, gsem0, ssem0), (rows1, idx1, gsem1, ssem1),
            (rows2, idx2, gsem2, ssem2)]
    NBUF = len(BUFS)

    for k in range(NBUF - 1):
        gather(k, BUFS[k][0], BUFS[k][1], BUFS[k][2])

    def stage(i, buf, gsem, ssem, nbuf, nibuf, ngsem, nssem):
        gwait(buf, gsem)

        @pl.when(i > 0)
        def _():
            acc_swait(i - 1, nbuf, nssem)

        @pl.when(i + NBUF - 1 < N_FULL)
        def _():
            gather(i + NBUF - 1, nbuf, nibuf, ngsem)

        acc_scatter(i, buf, ssem)

        for g in range(CHUNK // 16):
            idx = dst_all[pl.ds(i * CHUNK + g * 16, 16)]
            plsc.addupdate_scatter(hist, [idx], one16)

    def nbuf_chunks(j, carry):
        for k in range(NBUF):
            buf, _, gsem, ssem = BUFS[k]
            nbuf, nibuf, ngsem, nssem = BUFS[(k + NBUF - 1) % NBUF]
            stage(NBUF * j + k, buf, gsem, ssem, nbuf, nibuf, ngsem, nssem)
        return carry

    lax.fori_loop(0, N_FULL // NBUF, nbuf_chunks, 0)
    acc_swait(N_FULL - 1, rows2, ssem2)

    for g in range(TAIL // 16):
        v = src_all[pl.ds(N_FULL * CHUNK + g * 16, 16)]
        idx0[pl.ds(g * 16, 16)] = v + v + cid
    tidx = idx0.at[pl.ds(0, TAIL)]
    tbuf = rows0.at[pl.ds(0, TAIL)]
    pltpu.sync_copy(xv_hbm.at[tidx], tbuf)

    tdidx = dst_all.at[pl.ds(N_FULL * CHUNK, TAIL)]
    pltpu.sync_copy(tbuf, acc_sh.at[tdidx], add=True)

    for g in range(TAIL // 16):
        idx = dst_all[pl.ds(N_FULL * CHUNK + g * 16, 16)]
        plsc.addupdate_scatter(hist, [idx], one16)

    @pl.when(cid == 0)
    def _():
        pltpu.sync_copy(hist, cnt_out.at[sid])

    plsc.subcore_barrier()

    rbase = sid * ROWS_PER_TILE
    pltpu.sync_copy(acc_sh.at[pl.ds(rbase, ROWS_PER_TILE)],
                    acc_out.at[cid, pl.ds(rbase, ROWS_PER_TILE)])


@jax.jit
def _sc_aggregate(xv, edge_index):
    mesh = plsc.VectorSubcoreMesh(core_axis_name="c", subcore_axis_name="s")
    f = pl.kernel(
        _sc_kernel,
        out_type=[
            jax.ShapeDtypeStruct((NC, N_PAD, DH), jnp.float32),
            jax.ShapeDtypeStruct((NS, N_PAD), jnp.float32),
        ],
        mesh=mesh,
        scratch_types=[
            pltpu.VMEM((E_PER_S,), jnp.int32),
            pltpu.VMEM((E_PER_S,), jnp.int32),
            pltpu.VMEM((CHUNK,), jnp.int32),
            pltpu.VMEM((CHUNK,), jnp.int32),
            pltpu.VMEM((CHUNK,), jnp.int32),
            pltpu.VMEM((CHUNK, DH), jnp.float32),
            pltpu.VMEM((CHUNK, DH), jnp.float32),
            pltpu.VMEM((CHUNK, DH), jnp.float32),
            pltpu.VMEM((128, DH), jnp.float32),
            pltpu.VMEM((N_PAD,), jnp.float32),
            pltpu.VMEM_SHARED((N_PAD, DH), jnp.float32),
            pltpu.SemaphoreType.DMA,
            pltpu.SemaphoreType.DMA,
            pltpu.SemaphoreType.DMA,
            pltpu.SemaphoreType.DMA,
            pltpu.SemaphoreType.DMA,
            pltpu.SemaphoreType.DMA,
        ],
        compiler_params=pltpu.CompilerParams(use_tc_tiling_on_sc=False,
                                             needs_layout_passes=False),
    )
    return f(xv, edge_index)


def _tc_finish_body(acc_ref, cnt_ref, w_ref, o_ref):
    s = jnp.concatenate([acc_ref[0], acc_ref[1]], axis=1)
    c = jnp.sum(cnt_ref[...], axis=0)
    z = lax.dot_general(s, w_ref[...], (((1,), (1,)), ((), ())),
                        preferred_element_type=jnp.float32)
    o_ref[...] = z / jnp.maximum(c, 1.0)[:, None]


@jax.jit
def _tc_finish(acc, cnt, W):
    blk = 1024
    return pl.pallas_call(
        _tc_finish_body,
        grid=(N_PAD // blk,),
        in_specs=[
            pl.BlockSpec((NC, blk, DH), lambda i: (0, i, 0)),
            pl.BlockSpec((NS, blk), lambda i: (0, i)),
            pl.BlockSpec((D, D), lambda i: (0, 0)),
        ],
        out_specs=pl.BlockSpec((blk, D), lambda i: (i, 0)),
        out_shape=jax.ShapeDtypeStruct((N_PAD, D), jnp.float32),
    )(acc, cnt, W)


def kernel(x, edge_index, W):
    acc, cnt = _sc_aggregate(x.reshape(2 * N_NODES, DH), edge_index)
    return _tc_finish(acc, cnt, W)[:N_NODES]
